# Initial kernel scaffold; baseline (speedup 1.0000x reference)
import jax
import jax.numpy as jnp
from jax.experimental import pallas as pl


def kernel(g, inputs, edge_weights, W_res, b_res, W1, b1, W2, b2, W3, b3, W4, b4, W_op, b_op):
    raise NotImplementedError("write your pallas kernel here")


# speedup vs baseline: 6.1683x; 6.1683x over previous
import functools

import jax
import jax.numpy as jnp
from jax import lax
from jax.experimental import pallas as pl
from jax.experimental.pallas import tpu as pltpu
from jax.experimental.pallas import tpu_sc as plsc

f32 = jnp.float32
i32 = jnp.int32

N = 10000
D = 128
NP = 10240
CH = 128
GTOT = 2560
EP = GTOT * CH
NSUB = 16
NCORE = 2
NW = NSUB * NCORE
GPW = GTOT // NW
GPS = GTOT // NSUB
RPT = NP // NSUB


def _zeros16():
  return jnp.zeros((16,), f32)


def _deg_body(src_hbm, dst_hbm, deg_hbm, dout_sh, din_sh, sidx, didx,
              ones_v, zv, dsem):
  cid = lax.axis_index("c")
  sid = lax.axis_index("s")
  wid = sid * NCORE + cid

  def _init(r, c):
    zv[pl.ds(r * 16, 16)] = _zeros16()
    return c
  lax.fori_loop(0, RPT // 16, _init, 0)
  for r in range(CH // 16):
    ones_v[pl.ds(r * 16, 16)] = _zeros16() + 1.0

  pltpu.sync_copy(zv, dout_sh.at[pl.ds(sid * RPT, RPT)])
  pltpu.sync_copy(zv, din_sh.at[pl.ds(sid * RPT, RPT)])
  plsc.subcore_barrier()

  def _deg(j, c):
    gsl = pl.ds(wid * GPW + j * 8, 8)
    pltpu.sync_copy(src_hbm.at[gsl], sidx)
    pltpu.sync_copy(dst_hbm.at[gsl], didx)
    ds_ = []
    for i in range(8):
      ds_.append(pltpu.async_copy(ones_v, dout_sh.at[sidx.at[i]], dsem,
                                  add=True))
      ds_.append(pltpu.async_copy(ones_v, din_sh.at[didx.at[i]], dsem,
                                  add=True))
    for d in ds_:
      d.wait()
    return c
  lax.fori_loop(0, GPW // 8, _deg, 0)
  plsc.subcore_barrier()

  sl = pl.ds(sid * RPT, RPT)
  pltpu.sync_copy(dout_sh.at[sl], zv)
  pltpu.sync_copy(zv, deg_hbm.at[cid].at[0].at[sl])
  pltpu.sync_copy(din_sh.at[sl], zv)
  pltpu.sync_copy(zv, deg_hbm.at[cid].at[1].at[sl])


def _deg_call(srcp, dstp):
  mesh = plsc.VectorSubcoreMesh(core_axis_name="c", subcore_axis_name="s")
  fn = pl.kernel(
      _deg_body,
      out_type=jax.ShapeDtypeStruct((NCORE, 2, NP), f32),
      mesh=mesh,
      compiler_params=pltpu.CompilerParams(needs_layout_passes=False),
      scratch_types=[
          pltpu.VMEM_SHARED((NP,), f32),
          pltpu.VMEM_SHARED((NP,), f32),
          pltpu.VMEM((8, CH), i32),
          pltpu.VMEM((8, CH), i32),
          pltpu.VMEM((CH,), f32),
          pltpu.VMEM((RPT,), f32),
          pltpu.SemaphoreType.DMA,
      ],
  )
  return fn(srcp, dstp)


def _rsqrt_kernel(d_ref, o_ref):
  d = d_ref[...]
  s = jnp.maximum(d[0:2] + d[2:4], 1.0)
  o_ref[...] = lax.rsqrt(s)


def _rsqrt_call(deg):
  return pl.pallas_call(
      _rsqrt_kernel,
      out_shape=jax.ShapeDtypeStruct((2, NP), f32),
  )(deg.reshape(4, NP))


def _coef_body(src_hbm, dst_hbm, ew_hbm, dinv_hbm, w_hbm,
               dout_v, din_v, sidx, didx, ewb, wbuf):
  cid = lax.axis_index("c")
  sid = lax.axis_index("s")
  wid = sid * NCORE + cid

  pltpu.sync_copy(dinv_hbm.at[0], dout_v)
  pltpu.sync_copy(dinv_hbm.at[1], din_v)

  def _coef(k, c):
    gsl = pl.ds(wid * GPW + k * 8, 8)
    pltpu.sync_copy(src_hbm.at[gsl], sidx)
    pltpu.sync_copy(dst_hbm.at[gsl], didx)
    pltpu.sync_copy(ew_hbm.at[gsl], ewb)

    @plsc.parallel_loop(0, 8 * (CH // 16), step=1, unroll=4)
    def _cf(t):
      j = t // (CH // 16)
      sl = pl.ds((t % (CH // 16)) * 16, 16)
      a = plsc.load_gather(dout_v, [sidx[j, sl]])
      b = plsc.load_gather(din_v, [didx[j, sl]])
      wbuf[j, sl] = ewb[j, sl] * a * b
    pltpu.sync_copy(wbuf, w_hbm.at[gsl])
    return c
  lax.fori_loop(0, GPW // 8, _coef, 0)


def _coef_call(srcp, dstp, ewp, dinv):
  mesh = plsc.VectorSubcoreMesh(core_axis_name="c", subcore_axis_name="s")
  fn = pl.kernel(
      _coef_body,
      out_type=jax.ShapeDtypeStruct((GTOT, CH), f32),
      mesh=mesh,
      compiler_params=pltpu.CompilerParams(needs_layout_passes=False),
      scratch_types=[
          pltpu.VMEM((NP,), f32),
          pltpu.VMEM((NP,), f32),
          pltpu.VMEM((8, CH), i32),
          pltpu.VMEM((8, CH), i32),
          pltpu.VMEM((8, CH), f32),
          pltpu.VMEM((8, CH), f32),
      ],
  )
  return fn(srcp, dstp, ewp, dinv)


def _layer_body(x_hbm, src_hbm, dst_hbm, w_hbm, out_hbm,
                agg_sh, rows, sidx, didx, wv, gsem, ssem, isem):
  cid = lax.axis_index("c")
  sid = lax.axis_index("s")
  wid = sid * NCORE + cid
  g0 = wid * GPW

  def _z(r, c):
    for f in range(8):
      rows[0, r, pl.ds(f * 16, 16)] = _zeros16()
    return c
  lax.fori_loop(0, CH, _z, 0)
  for q in range(RPT // CH):
    pltpu.sync_copy(rows.at[0], agg_sh.at[pl.ds(sid * RPT + q * CH, CH)])
  plsc.subcore_barrier()

  def _load_idx(g, q):
    pltpu.sync_copy(src_hbm.at[g], sidx.at[q])
    pltpu.sync_copy(dst_hbm.at[g], didx.at[q])
    pltpu.sync_copy(w_hbm.at[g], wv.at[q])

  def _fire_gather(q, b):
    return pltpu.async_copy(x_hbm.at[sidx.at[q]], rows.at[b], gsem)

  def _wait_gather(b):
    pltpu.make_async_copy(x_hbm.at[sidx.at[0]], rows.at[b], gsem).wait()

  def _scale(b, q):
    @plsc.parallel_loop(0, CH, step=1, unroll=4)
    def _sc(e):
      wb = plsc.load_gather(wv.at[q], [jnp.zeros((16,), i32) + e])
      for f in range(8):
        sl = pl.ds(f * 16, 16)
        rows[b, e, sl] = rows[b, e, sl] * wb

  _load_idx(g0, 0)
  _fire_gather(0, 0)

  def _iter(k, c):
    b = k % 2
    nb = 1 - b
    _load_idx(g0 + jnp.minimum(k + 1, GPW - 1), nb)
    _fire_gather(nb, nb)
    _wait_gather(b)
    _scale(b, b)
    d = pltpu.async_copy(rows.at[b], agg_sh.at[didx.at[b]], ssem, add=True)
    d.wait()
    return c
  lax.fori_loop(0, GPW, _iter, 0)
  _wait_gather(0)
  plsc.subcore_barrier()

  for q in range(RPT // CH):
    sl = pl.ds(sid * RPT + q * CH, CH)
    pltpu.sync_copy(agg_sh.at[sl], rows.at[0])
    pltpu.sync_copy(rows.at[0], out_hbm.at[cid].at[sl])


def _layer_call(x, srcp, dstp, w2d):
  mesh = plsc.VectorSubcoreMesh(core_axis_name="c", subcore_axis_name="s")
  fn = pl.kernel(
      _layer_body,
      out_type=jax.ShapeDtypeStruct((NCORE, NP, D), f32),
      mesh=mesh,
      compiler_params=pltpu.CompilerParams(needs_layout_passes=False),
      scratch_types=[
          pltpu.VMEM_SHARED((NP, D), f32),
          pltpu.VMEM((2, CH, D), f32),
          pltpu.VMEM((4, CH), i32),
          pltpu.VMEM((4, CH), i32),
          pltpu.VMEM((4, CH), f32),
          pltpu.SemaphoreType.DMA,
          pltpu.SemaphoreType.DMA,
          pltpu.SemaphoreType.DMA,
      ],
  )
  return fn(x, srcp, dstp, w2d)


def _mm_relu_kernel(a0_ref, a1_ref, w_ref, b_ref, o_ref):
  acc = jnp.dot(a0_ref[...] + a1_ref[...], w_ref[...],
                preferred_element_type=f32)
  o_ref[...] = jnp.maximum(acc + b_ref[...], 0.0)


def _mm_layer(a0, a1, W, b):
  return pl.pallas_call(
      _mm_relu_kernel,
      grid=(NP // 512,),
      in_specs=[
          pl.BlockSpec((512, D), lambda i: (i, 0)),
          pl.BlockSpec((512, D), lambda i: (i, 0)),
          pl.BlockSpec((D, D), lambda i: (0, 0)),
          pl.BlockSpec((1, D), lambda i: (0, 0)),
      ],
      out_specs=pl.BlockSpec((512, D), lambda i: (i, 0)),
      out_shape=jax.ShapeDtypeStruct((NP, D), f32),
  )(a0, a1, W, b.reshape(1, D))


def _final_kernel(a0_ref, a1_ref, w4_ref, b4_ref, x0_ref, wr_ref, br_ref,
                  wo_ref, bo_ref, o_ref):
  t = jnp.dot(a0_ref[...] + a1_ref[...], w4_ref[...],
              preferred_element_type=f32) + b4_ref[...]
  t = t + jnp.dot(x0_ref[...], wr_ref[...],
                  preferred_element_type=f32) + br_ref[...]
  t = jnp.maximum(t, 0.0)
  o_ref[...] = jnp.dot(t, wo_ref[...], preferred_element_type=f32) + bo_ref[...]


def _final_call(a0, a1, W4, b4, x0, W_res, b_res, wo, bo):
  return pl.pallas_call(
      _final_kernel,
      grid=(NP // 512,),
      in_specs=[
          pl.BlockSpec((512, D), lambda i: (i, 0)),
          pl.BlockSpec((512, D), lambda i: (i, 0)),
          pl.BlockSpec((D, D), lambda i: (0, 0)),
          pl.BlockSpec((1, D), lambda i: (0, 0)),
          pl.BlockSpec((512, D), lambda i: (i, 0)),
          pl.BlockSpec((D, D), lambda i: (0, 0)),
          pl.BlockSpec((1, D), lambda i: (0, 0)),
          pl.BlockSpec((D, D), lambda i: (0, 0)),
          pl.BlockSpec((1, D), lambda i: (0, 0)),
      ],
      out_specs=pl.BlockSpec((512, D), lambda i: (i, 0)),
      out_shape=jax.ShapeDtypeStruct((NP, D), f32),
  )(a0, a1, W4, b4.reshape(1, D), x0, W_res, b_res.reshape(1, D), wo,
    bo.reshape(1, D))


def kernel(g, inputs, edge_weights, W_res, b_res, W1, b1, W2, b2, W3, b3,
           W4, b4, W_op, b_op):
  E = edge_weights.shape[0]
  pad = EP - E
  trash = N + jnp.arange(pad, dtype=i32) % (NP - N)
  src = jnp.concatenate([g[0].astype(i32), trash])
  dst = jnp.concatenate([g[1].astype(i32), trash])
  ew = jnp.concatenate([edge_weights.astype(f32), jnp.zeros((pad,), f32)])
  srcp = src.reshape(GTOT, CH)
  dstp = dst.reshape(GTOT, CH)
  ewp = ew.reshape(GTOT, CH)
  x0 = jnp.pad(inputs, ((0, NP - N), (0, 0)))

  deg = _deg_call(srcp, dstp)
  dinv = _rsqrt_call(deg)
  w2d = _coef_call(srcp, dstp, ewp, dinv)

  a = _layer_call(x0, srcp, dstp, w2d)
  x = _mm_layer(a[0], a[1], W1, b1)
  a = _layer_call(x, srcp, dstp, w2d)
  x = _mm_layer(a[0], a[1], W2, b2)
  a = _layer_call(x, srcp, dstp, w2d)
  x = _mm_layer(a[0], a[1], W3, b3)
  a = _layer_call(x, srcp, dstp, w2d)

  wo = jnp.pad(W_op, ((0, 0), (0, D - W_op.shape[1])))
  bo = jnp.pad(b_op, (0, D - b_op.shape[0]))
  out = _final_call(a[0], a[1], W4, b4, x0, W_res, b_res, wo, bo)
  return out[:N, :W_op.shape[1]]


# speedup vs baseline: 7.3758x; 1.1958x over previous
import functools

import jax
import jax.numpy as jnp
from jax import lax
from jax.experimental import pallas as pl
from jax.experimental.pallas import tpu as pltpu
from jax.experimental.pallas import tpu_sc as plsc

f32 = jnp.float32
i32 = jnp.int32

N = 10000
D = 128
NP = 10240
CH = 128
GTOT = 2560
EP = GTOT * CH
NSUB = 16
NCORE = 2
NW = NSUB * NCORE
GPW = GTOT // NW
GPS = GTOT // NSUB
RPT = NP // NSUB


def _zeros16():
  return jnp.zeros((16,), f32)


def _deg_body(src_hbm, dst_hbm, deg_hbm, dout_sh, din_sh, sidx, didx,
              ones_v, zv, dsem):
  cid = lax.axis_index("c")
  sid = lax.axis_index("s")
  wid = sid * NCORE + cid

  def _init(r, c):
    zv[pl.ds(r * 16, 16)] = _zeros16()
    return c
  lax.fori_loop(0, RPT // 16, _init, 0)
  for r in range(CH // 16):
    ones_v[pl.ds(r * 16, 16)] = _zeros16() + 1.0

  pltpu.sync_copy(zv, dout_sh.at[pl.ds(sid * RPT, RPT)])
  pltpu.sync_copy(zv, din_sh.at[pl.ds(sid * RPT, RPT)])
  plsc.subcore_barrier()

  def _deg(j, c):
    gsl = pl.ds(wid * GPW + j * 8, 8)
    pltpu.sync_copy(src_hbm.at[gsl], sidx)
    pltpu.sync_copy(dst_hbm.at[gsl], didx)
    ds_ = []
    for i in range(8):
      ds_.append(pltpu.async_copy(ones_v, dout_sh.at[sidx.at[i]], dsem,
                                  add=True))
      ds_.append(pltpu.async_copy(ones_v, din_sh.at[didx.at[i]], dsem,
                                  add=True))
    for d in ds_:
      d.wait()
    return c
  lax.fori_loop(0, GPW // 8, _deg, 0)
  plsc.subcore_barrier()

  sl = pl.ds(sid * RPT, RPT)
  pltpu.sync_copy(dout_sh.at[sl], zv)
  pltpu.sync_copy(zv, deg_hbm.at[cid].at[0].at[sl])
  pltpu.sync_copy(din_sh.at[sl], zv)
  pltpu.sync_copy(zv, deg_hbm.at[cid].at[1].at[sl])


def _deg_call(srcp, dstp):
  mesh = plsc.VectorSubcoreMesh(core_axis_name="c", subcore_axis_name="s")
  fn = pl.kernel(
      _deg_body,
      out_type=jax.ShapeDtypeStruct((NCORE, 2, NP), f32),
      mesh=mesh,
      compiler_params=pltpu.CompilerParams(needs_layout_passes=False),
      scratch_types=[
          pltpu.VMEM_SHARED((NP,), f32),
          pltpu.VMEM_SHARED((NP,), f32),
          pltpu.VMEM((8, CH), i32),
          pltpu.VMEM((8, CH), i32),
          pltpu.VMEM((CH,), f32),
          pltpu.VMEM((RPT,), f32),
          pltpu.SemaphoreType.DMA,
      ],
  )
  return fn(srcp, dstp)


def _rsqrt_kernel(d_ref, o_ref):
  d = d_ref[...]
  s = jnp.maximum(d[0:2] + d[2:4], 1.0)
  o_ref[...] = lax.rsqrt(s)


def _rsqrt_call(deg):
  return pl.pallas_call(
      _rsqrt_kernel,
      out_shape=jax.ShapeDtypeStruct((2, NP), f32),
  )(deg.reshape(4, NP))


def _coef_body(src_hbm, dst_hbm, ew_hbm, dinv_hbm, w_hbm,
               dout_v, din_v, sidx, didx, ewb, wbuf):
  cid = lax.axis_index("c")
  sid = lax.axis_index("s")
  wid = sid * NCORE + cid

  pltpu.sync_copy(dinv_hbm.at[0], dout_v)
  pltpu.sync_copy(dinv_hbm.at[1], din_v)

  def _coef(k, c):
    gsl = pl.ds(wid * GPW + k * 8, 8)
    pltpu.sync_copy(src_hbm.at[gsl], sidx)
    pltpu.sync_copy(dst_hbm.at[gsl], didx)
    pltpu.sync_copy(ew_hbm.at[gsl], ewb)

    @plsc.parallel_loop(0, 8 * (CH // 16), step=1, unroll=4)
    def _cf(t):
      j = t // (CH // 16)
      sl = pl.ds((t % (CH // 16)) * 16, 16)
      a = plsc.load_gather(dout_v, [sidx[j, sl]])
      b = plsc.load_gather(din_v, [didx[j, sl]])
      wbuf[j, sl] = ewb[j, sl] * a * b
    pltpu.sync_copy(wbuf, w_hbm.at[gsl])
    return c
  lax.fori_loop(0, GPW // 8, _coef, 0)


def _coef_call(srcp, dstp, ewp, dinv):
  mesh = plsc.VectorSubcoreMesh(core_axis_name="c", subcore_axis_name="s")
  fn = pl.kernel(
      _coef_body,
      out_type=jax.ShapeDtypeStruct((GTOT, CH), f32),
      mesh=mesh,
      compiler_params=pltpu.CompilerParams(needs_layout_passes=False),
      scratch_types=[
          pltpu.VMEM((NP,), f32),
          pltpu.VMEM((NP,), f32),
          pltpu.VMEM((8, CH), i32),
          pltpu.VMEM((8, CH), i32),
          pltpu.VMEM((8, CH), f32),
          pltpu.VMEM((8, CH), f32),
      ],
  )
  return fn(srcp, dstp, ewp, dinv)


def _layer_body(x_hbm, src_hbm, dst_hbm, w_hbm, out_hbm,
                agg_sh, rows, sidx, didx, wv, gsem, ssem, isem):
  cid = lax.axis_index("c")
  sid = lax.axis_index("s")
  wid = sid * NCORE + cid
  g0 = wid * GPW

  def _z(r, c):
    for f in range(8):
      rows[0, r, pl.ds(f * 16, 16)] = _zeros16()
    return c
  lax.fori_loop(0, CH, _z, 0)
  for q in range(RPT // CH):
    pltpu.sync_copy(rows.at[0], agg_sh.at[pl.ds(sid * RPT + q * CH, CH)])
  plsc.subcore_barrier()

  def _load_idx(g, q):
    pltpu.sync_copy(src_hbm.at[g], sidx.at[q])
    pltpu.sync_copy(dst_hbm.at[g], didx.at[q])
    pltpu.sync_copy(w_hbm.at[g], wv.at[q])

  def _fire_gather(q, b):
    return pltpu.async_copy(x_hbm.at[sidx.at[q]], rows.at[b], gsem)

  def _wait_gather(b):
    pltpu.make_async_copy(x_hbm.at[sidx.at[0]], rows.at[b], gsem).wait()

  def _scale(b, q):
    @plsc.parallel_loop(0, CH, step=1, unroll=4)
    def _sc(e):
      wb = plsc.load_gather(wv.at[q], [jnp.zeros((16,), i32) + e])
      for f in range(8):
        sl = pl.ds(f * 16, 16)
        rows[b, e, sl] = rows[b, e, sl] * wb

  def _wait_scatter(q, b):
    pltpu.make_async_copy(rows.at[b], agg_sh.at[didx.at[q]], ssem).wait()

  _load_idx(g0, 0)
  _fire_gather(0, 0)
  _load_idx(g0 + 1, 1)
  _fire_gather(1, 1)
  _wait_gather(0)
  _scale(0, 0)
  d0 = pltpu.async_copy(rows.at[0], agg_sh.at[didx.at[0]], ssem, add=True)

  def _iter(k, c):
    b = k % 2
    nb = 1 - b
    q = k % 3
    _load_idx(g0 + jnp.minimum(k + 1, GPW - 1), (k + 1) % 3)
    _wait_gather(b)
    _wait_scatter((k - 1) % 3, nb)
    _fire_gather((k + 1) % 3, nb)
    _scale(b, q)
    pltpu.async_copy(rows.at[b], agg_sh.at[didx.at[q]], ssem, add=True)
    return c
  lax.fori_loop(1, GPW, _iter, 0)
  b_last = (GPW - 1) % 2
  _wait_gather(1 - b_last)
  _wait_scatter((GPW - 1) % 3, b_last)
  plsc.subcore_barrier()

  for q in range(RPT // CH):
    sl = pl.ds(sid * RPT + q * CH, CH)
    pltpu.sync_copy(agg_sh.at[sl], rows.at[0])
    pltpu.sync_copy(rows.at[0], out_hbm.at[cid].at[sl])


def _layer_call(x, srcp, dstp, w2d):
  mesh = plsc.VectorSubcoreMesh(core_axis_name="c", subcore_axis_name="s")
  fn = pl.kernel(
      _layer_body,
      out_type=jax.ShapeDtypeStruct((NCORE, NP, D), f32),
      mesh=mesh,
      compiler_params=pltpu.CompilerParams(needs_layout_passes=False),
      scratch_types=[
          pltpu.VMEM_SHARED((NP, D), f32),
          pltpu.VMEM((2, CH, D), f32),
          pltpu.VMEM((4, CH), i32),
          pltpu.VMEM((4, CH), i32),
          pltpu.VMEM((4, CH), f32),
          pltpu.SemaphoreType.DMA,
          pltpu.SemaphoreType.DMA,
          pltpu.SemaphoreType.DMA,
      ],
  )
  return fn(x, srcp, dstp, w2d)


def _mm_relu_kernel(a0_ref, a1_ref, w_ref, b_ref, o_ref):
  acc = jnp.dot(a0_ref[...] + a1_ref[...], w_ref[...],
                preferred_element_type=f32)
  o_ref[...] = jnp.maximum(acc + b_ref[...], 0.0)


def _mm_layer(a0, a1, W, b):
  return pl.pallas_call(
      _mm_relu_kernel,
      grid=(NP // 512,),
      in_specs=[
          pl.BlockSpec((512, D), lambda i: (i, 0)),
          pl.BlockSpec((512, D), lambda i: (i, 0)),
          pl.BlockSpec((D, D), lambda i: (0, 0)),
          pl.BlockSpec((1, D), lambda i: (0, 0)),
      ],
      out_specs=pl.BlockSpec((512, D), lambda i: (i, 0)),
      out_shape=jax.ShapeDtypeStruct((NP, D), f32),
  )(a0, a1, W, b.reshape(1, D))


def _final_kernel(a0_ref, a1_ref, w4_ref, b4_ref, x0_ref, wr_ref, br_ref,
                  wo_ref, bo_ref, o_ref):
  t = jnp.dot(a0_ref[...] + a1_ref[...], w4_ref[...],
              preferred_element_type=f32) + b4_ref[...]
  t = t + jnp.dot(x0_ref[...], wr_ref[...],
                  preferred_element_type=f32) + br_ref[...]
  t = jnp.maximum(t, 0.0)
  o_ref[...] = jnp.dot(t, wo_ref[...], preferred_element_type=f32) + bo_ref[...]


def _final_call(a0, a1, W4, b4, x0, W_res, b_res, wo, bo):
  return pl.pallas_call(
      _final_kernel,
      grid=(NP // 512,),
      in_specs=[
          pl.BlockSpec((512, D), lambda i: (i, 0)),
          pl.BlockSpec((512, D), lambda i: (i, 0)),
          pl.BlockSpec((D, D), lambda i: (0, 0)),
          pl.BlockSpec((1, D), lambda i: (0, 0)),
          pl.BlockSpec((512, D), lambda i: (i, 0)),
          pl.BlockSpec((D, D), lambda i: (0, 0)),
          pl.BlockSpec((1, D), lambda i: (0, 0)),
          pl.BlockSpec((D, D), lambda i: (0, 0)),
          pl.BlockSpec((1, D), lambda i: (0, 0)),
      ],
      out_specs=pl.BlockSpec((512, D), lambda i: (i, 0)),
      out_shape=jax.ShapeDtypeStruct((NP, D), f32),
  )(a0, a1, W4, b4.reshape(1, D), x0, W_res, b_res.reshape(1, D), wo,
    bo.reshape(1, D))


def kernel(g, inputs, edge_weights, W_res, b_res, W1, b1, W2, b2, W3, b3,
           W4, b4, W_op, b_op):
  E = edge_weights.shape[0]
  pad = EP - E
  trash = N + jnp.arange(pad, dtype=i32) % (NP - N)
  src = jnp.concatenate([g[0].astype(i32), trash])
  dst = jnp.concatenate([g[1].astype(i32), trash])
  ew = jnp.concatenate([edge_weights.astype(f32), jnp.zeros((pad,), f32)])
  srcp = src.reshape(GTOT, CH)
  dstp = dst.reshape(GTOT, CH)
  ewp = ew.reshape(GTOT, CH)
  x0 = jnp.pad(inputs, ((0, NP - N), (0, 0)))

  deg = _deg_call(srcp, dstp)
  dinv = _rsqrt_call(deg)
  w2d = _coef_call(srcp, dstp, ewp, dinv)

  a = _layer_call(x0, srcp, dstp, w2d)
  x = _mm_layer(a[0], a[1], W1, b1)
  a = _layer_call(x, srcp, dstp, w2d)
  x = _mm_layer(a[0], a[1], W2, b2)
  a = _layer_call(x, srcp, dstp, w2d)
  x = _mm_layer(a[0], a[1], W3, b3)
  a = _layer_call(x, srcp, dstp, w2d)

  wo = jnp.pad(W_op, ((0, 0), (0, D - W_op.shape[1])))
  bo = jnp.pad(b_op, (0, D - b_op.shape[0]))
  out = _final_call(a[0], a[1], W4, b4, x0, W_res, b_res, wo, bo)
  return out[:N, :W_op.shape[1]]


# speedup vs baseline: 9.6334x; 1.3061x over previous
import functools

import jax
import jax.numpy as jnp
from jax import lax
from jax.experimental import pallas as pl
from jax.experimental.pallas import tpu as pltpu
from jax.experimental.pallas import tpu_sc as plsc

f32 = jnp.float32
i32 = jnp.int32

N = 10000
D = 128
NP = 10240
CH = 128
GTOT = 2560
EP = GTOT * CH
NSUB = 16
NCORE = 2
NW = NSUB * NCORE
GPW = GTOT // NW
GPS = GTOT // NSUB
RPT = NP // NSUB


def _zeros16():
  return jnp.zeros((16,), f32)


def _deg_body(src_hbm, dst_hbm, deg_hbm, dout_sh, din_sh, sidx, didx,
              ones_v, zv, dsem):
  cid = lax.axis_index("c")
  sid = lax.axis_index("s")
  wid = sid * NCORE + cid

  def _init(r, c):
    zv[pl.ds(r * 16, 16)] = _zeros16()
    return c
  lax.fori_loop(0, RPT // 16, _init, 0)
  for r in range(CH // 16):
    ones_v[pl.ds(r * 16, 16)] = _zeros16() + 1.0

  pltpu.sync_copy(zv, dout_sh.at[pl.ds(sid * RPT, RPT)])
  pltpu.sync_copy(zv, din_sh.at[pl.ds(sid * RPT, RPT)])
  plsc.subcore_barrier()

  def _deg(j, c):
    gsl = pl.ds(wid * GPW + j * 8, 8)
    pltpu.sync_copy(src_hbm.at[gsl], sidx)
    pltpu.sync_copy(dst_hbm.at[gsl], didx)
    ds_ = []
    for i in range(8):
      ds_.append(pltpu.async_copy(ones_v, dout_sh.at[sidx.at[i]], dsem,
                                  add=True))
      ds_.append(pltpu.async_copy(ones_v, din_sh.at[didx.at[i]], dsem,
                                  add=True))
    for d in ds_:
      d.wait()
    return c
  lax.fori_loop(0, GPW // 8, _deg, 0)
  plsc.subcore_barrier()

  sl = pl.ds(sid * RPT, RPT)
  pltpu.sync_copy(dout_sh.at[sl], zv)
  pltpu.sync_copy(zv, deg_hbm.at[cid].at[0].at[sl])
  pltpu.sync_copy(din_sh.at[sl], zv)
  pltpu.sync_copy(zv, deg_hbm.at[cid].at[1].at[sl])


def _deg_call(srcp, dstp):
  mesh = plsc.VectorSubcoreMesh(core_axis_name="c", subcore_axis_name="s")
  fn = pl.kernel(
      _deg_body,
      out_type=jax.ShapeDtypeStruct((NCORE, 2, NP), f32),
      mesh=mesh,
      compiler_params=pltpu.CompilerParams(needs_layout_passes=False),
      scratch_types=[
          pltpu.VMEM_SHARED((NP,), f32),
          pltpu.VMEM_SHARED((NP,), f32),
          pltpu.VMEM((8, CH), i32),
          pltpu.VMEM((8, CH), i32),
          pltpu.VMEM((CH,), f32),
          pltpu.VMEM((RPT,), f32),
          pltpu.SemaphoreType.DMA,
      ],
  )
  return fn(srcp, dstp)


def _rsqrt_kernel(d_ref, o_ref):
  d = d_ref[...]
  s = jnp.maximum(d[0:2] + d[2:4], 1.0)
  o_ref[...] = lax.rsqrt(s)


def _rsqrt_call(deg):
  return pl.pallas_call(
      _rsqrt_kernel,
      out_shape=jax.ShapeDtypeStruct((2, NP), f32),
  )(deg.reshape(4, NP))


def _coef_body(src_hbm, dst_hbm, ew_hbm, dinv_hbm, w_hbm,
               dout_v, din_v, sidx, didx, ewb, wbuf):
  cid = lax.axis_index("c")
  sid = lax.axis_index("s")
  wid = sid * NCORE + cid

  pltpu.sync_copy(dinv_hbm.at[0], dout_v)
  pltpu.sync_copy(dinv_hbm.at[1], din_v)

  def _coef(k, c):
    gsl = pl.ds(wid * GPW + k * 8, 8)
    pltpu.sync_copy(src_hbm.at[gsl], sidx)
    pltpu.sync_copy(dst_hbm.at[gsl], didx)
    pltpu.sync_copy(ew_hbm.at[gsl], ewb)

    @plsc.parallel_loop(0, 8 * (CH // 16), step=1, unroll=4)
    def _cf(t):
      j = t // (CH // 16)
      sl = pl.ds((t % (CH // 16)) * 16, 16)
      a = plsc.load_gather(dout_v, [sidx[j, sl]])
      b = plsc.load_gather(din_v, [didx[j, sl]])
      wbuf[j, sl] = ewb[j, sl] * a * b
    pltpu.sync_copy(wbuf, w_hbm.at[gsl])
    return c
  lax.fori_loop(0, GPW // 8, _coef, 0)


def _coef_call(srcp, dstp, ewp, dinv):
  mesh = plsc.VectorSubcoreMesh(core_axis_name="c", subcore_axis_name="s")
  fn = pl.kernel(
      _coef_body,
      out_type=jax.ShapeDtypeStruct((GTOT, CH), f32),
      mesh=mesh,
      compiler_params=pltpu.CompilerParams(needs_layout_passes=False),
      scratch_types=[
          pltpu.VMEM((NP,), f32),
          pltpu.VMEM((NP,), f32),
          pltpu.VMEM((8, CH), i32),
          pltpu.VMEM((8, CH), i32),
          pltpu.VMEM((8, CH), f32),
          pltpu.VMEM((8, CH), f32),
      ],
  )
  return fn(srcp, dstp, ewp, dinv)


def _layer_body(x_hbm, src_hbm, dst_hbm, w_hbm, out_hbm,
                agg_sh, rows, sidx, didx, wv, gsem, ssem, isem):
  cid = lax.axis_index("c")
  sid = lax.axis_index("s")
  wid = sid * NCORE + cid
  g0 = wid * GPW

  def _z(r, c):
    for f in range(8):
      rows[0, r, pl.ds(f * 16, 16)] = _zeros16()
    return c
  lax.fori_loop(0, CH, _z, 0)
  zd = [pltpu.async_copy(rows.at[0],
                         agg_sh.at[pl.ds(sid * RPT + q * CH, CH)], isem)
        for q in range(RPT // CH)]
  for d in zd:
    d.wait()
  plsc.subcore_barrier()

  def _load_idx(g, q):
    ds = [pltpu.async_copy(src_hbm.at[g], sidx.at[q], isem),
          pltpu.async_copy(dst_hbm.at[g], didx.at[q], isem),
          pltpu.async_copy(w_hbm.at[g], wv.at[q], isem)]
    for d in ds:
      d.wait()

  def _fire_gather(q, b):
    return pltpu.async_copy(x_hbm.at[sidx.at[q]], rows.at[b], gsem)

  def _wait_gather(b):
    pltpu.make_async_copy(x_hbm.at[sidx.at[0]], rows.at[b], gsem).wait()

  def _scale(b, q):
    @plsc.parallel_loop(0, CH, step=1, unroll=4)
    def _sc(e):
      wb = plsc.load_gather(wv.at[q], [jnp.zeros((16,), i32) + e])
      for f in range(8):
        sl = pl.ds(f * 16, 16)
        rows[b, e, sl] = rows[b, e, sl] * wb

  def _wait_scatter(q, b):
    pltpu.make_async_copy(rows.at[b], agg_sh.at[didx.at[q]], ssem).wait()

  _load_idx(g0, 0)
  _fire_gather(0, 0)
  _load_idx(g0 + 1, 1)
  _fire_gather(1, 1)
  _wait_gather(0)
  _scale(0, 0)
  d0 = pltpu.async_copy(rows.at[0], agg_sh.at[didx.at[0]], ssem, add=True)

  def _iter(k, c):
    b = k % 2
    nb = 1 - b
    q = k % 3
    _load_idx(g0 + jnp.minimum(k + 1, GPW - 1), (k + 1) % 3)
    _wait_gather(b)
    _wait_scatter((k - 1) % 3, nb)
    _fire_gather((k + 1) % 3, nb)
    _scale(b, q)
    pltpu.async_copy(rows.at[b], agg_sh.at[didx.at[q]], ssem, add=True)
    return c
  lax.fori_loop(1, GPW, _iter, 0)
  b_last = (GPW - 1) % 2
  _wait_gather(1 - b_last)
  _wait_scatter((GPW - 1) % 3, b_last)
  plsc.subcore_barrier()

  def _dsl(q):
    return pl.ds(sid * RPT + q * CH, CH)
  hs = [None, None]
  pltpu.sync_copy(agg_sh.at[_dsl(0)], rows.at[0])
  for q in range(RPT // CH):
    hs[q % 2] = pltpu.async_copy(rows.at[q % 2], out_hbm.at[cid].at[_dsl(q)],
                                 gsem)
    if q + 1 < RPT // CH:
      if hs[(q + 1) % 2] is not None:
        hs[(q + 1) % 2].wait()
      pltpu.sync_copy(agg_sh.at[_dsl(q + 1)], rows.at[(q + 1) % 2])
  hs[0].wait()
  hs[1].wait()


def _layer_call(x, srcp, dstp, w2d):
  mesh = plsc.VectorSubcoreMesh(core_axis_name="c", subcore_axis_name="s")
  fn = pl.kernel(
      _layer_body,
      out_type=jax.ShapeDtypeStruct((NCORE, NP, D), f32),
      mesh=mesh,
      compiler_params=pltpu.CompilerParams(needs_layout_passes=False),
      scratch_types=[
          pltpu.VMEM_SHARED((NP, D), f32),
          pltpu.VMEM((2, CH, D), f32),
          pltpu.VMEM((4, CH), i32),
          pltpu.VMEM((4, CH), i32),
          pltpu.VMEM((4, CH), f32),
          pltpu.SemaphoreType.DMA,
          pltpu.SemaphoreType.DMA,
          pltpu.SemaphoreType.DMA,
      ],
  )
  return fn(x, srcp, dstp, w2d)


def _mm_relu_kernel(a0_ref, a1_ref, w_ref, b_ref, o_ref):
  acc = jnp.dot(a0_ref[...] + a1_ref[...], w_ref[...],
                preferred_element_type=f32)
  o_ref[...] = jnp.maximum(acc + b_ref[...], 0.0)


def _mm_layer(a0, a1, W, b):
  return pl.pallas_call(
      _mm_relu_kernel,
      grid=(NP // 512,),
      in_specs=[
          pl.BlockSpec((512, D), lambda i: (i, 0)),
          pl.BlockSpec((512, D), lambda i: (i, 0)),
          pl.BlockSpec((D, D), lambda i: (0, 0)),
          pl.BlockSpec((1, D), lambda i: (0, 0)),
      ],
      out_specs=pl.BlockSpec((512, D), lambda i: (i, 0)),
      out_shape=jax.ShapeDtypeStruct((NP, D), f32),
  )(a0, a1, W, b.reshape(1, D))


def _final_kernel(a0_ref, a1_ref, w4_ref, b4_ref, x0_ref, wr_ref, br_ref,
                  wo_ref, bo_ref, o_ref):
  t = jnp.dot(a0_ref[...] + a1_ref[...], w4_ref[...],
              preferred_element_type=f32) + b4_ref[...]
  t = t + jnp.dot(x0_ref[...], wr_ref[...],
                  preferred_element_type=f32) + br_ref[...]
  t = jnp.maximum(t, 0.0)
  o_ref[...] = jnp.dot(t, wo_ref[...], preferred_element_type=f32) + bo_ref[...]


def _final_call(a0, a1, W4, b4, x0, W_res, b_res, wo, bo):
  return pl.pallas_call(
      _final_kernel,
      grid=(NP // 512,),
      in_specs=[
          pl.BlockSpec((512, D), lambda i: (i, 0)),
          pl.BlockSpec((512, D), lambda i: (i, 0)),
          pl.BlockSpec((D, D), lambda i: (0, 0)),
          pl.BlockSpec((1, D), lambda i: (0, 0)),
          pl.BlockSpec((512, D), lambda i: (i, 0)),
          pl.BlockSpec((D, D), lambda i: (0, 0)),
          pl.BlockSpec((1, D), lambda i: (0, 0)),
          pl.BlockSpec((D, D), lambda i: (0, 0)),
          pl.BlockSpec((1, D), lambda i: (0, 0)),
      ],
      out_specs=pl.BlockSpec((512, D), lambda i: (i, 0)),
      out_shape=jax.ShapeDtypeStruct((NP, D), f32),
  )(a0, a1, W4, b4.reshape(1, D), x0, W_res, b_res.reshape(1, D), wo,
    bo.reshape(1, D))


def kernel(g, inputs, edge_weights, W_res, b_res, W1, b1, W2, b2, W3, b3,
           W4, b4, W_op, b_op):
  E = edge_weights.shape[0]
  pad = EP - E
  trash = N + jnp.arange(pad, dtype=i32) % (NP - N)
  src = jnp.concatenate([g[0].astype(i32), trash])
  dst = jnp.concatenate([g[1].astype(i32), trash])
  ew = jnp.concatenate([edge_weights.astype(f32), jnp.zeros((pad,), f32)])
  srcp = src.reshape(GTOT, CH)
  dstp = dst.reshape(GTOT, CH)
  ewp = ew.reshape(GTOT, CH)
  x0 = jnp.pad(inputs, ((0, NP - N), (0, 0)))

  deg = _deg_call(srcp, dstp)
  dinv = _rsqrt_call(deg)
  w2d = _coef_call(srcp, dstp, ewp, dinv)

  a = _layer_call(x0, srcp, dstp, w2d)
  x = _mm_layer(a[0], a[1], W1, b1)
  a = _layer_call(x, srcp, dstp, w2d)
  x = _mm_layer(a[0], a[1], W2, b2)
  a = _layer_call(x, srcp, dstp, w2d)
  x = _mm_layer(a[0], a[1], W3, b3)
  a = _layer_call(x, srcp, dstp, w2d)

  wo = jnp.pad(W_op, ((0, 0), (0, D - W_op.shape[1])))
  bo = jnp.pad(b_op, (0, D - b_op.shape[0]))
  out = _final_call(a[0], a[1], W4, b4, x0, W_res, b_res, wo, bo)
  return out[:N, :W_op.shape[1]]
